# baseline (speedup 1.0000x reference)
import functools

import jax
import jax.numpy as jnp
from jax import lax
from jax.experimental import layout as jax_layout
from jax.experimental import pallas as pl
from jax.experimental.pallas import tpu as pltpu
from jax.experimental.pallas import tpu_sc as plsc

_B = 16384
_S = 50
_SP = 128
_D = 64
_NC = 2
_NS = 16
_NW = _NC * _NS
_ROWS_PER_W = _B // _NW
_IDX_PER_W = _ROWS_PER_W * _S
_STAGE_ROWS = 128
_N_STAGE = _ROWS_PER_W // _STAGE_ROWS
_CHUNK_ROWS = 4
_CHUNK = _CHUNK_ROWS * _S
_N_CHUNKS = _ROWS_PER_W // _CHUNK_ROWS
_NBUF = 2
_TP = 128
_L = 16


def _make_gather():
  mesh = plsc.VectorSubcoreMesh(core_axis_name="c", subcore_axis_name="s")

  @functools.partial(
      pl.kernel,
      out_type=jax.ShapeDtypeStruct((_B, _S, _D), jnp.float32),
      mesh=mesh,
      compiler_params=pltpu.CompilerParams(use_tc_tiling_on_sc=False),
      scratch_types=[
          pltpu.VMEM((_STAGE_ROWS, _SP), jnp.int32),
          pltpu.VMEM((_IDX_PER_W,), jnp.int32),
          pltpu.VMEM((_NBUF, _CHUNK, _TP), jnp.float32),
          [pltpu.SemaphoreType.DMA] * _NBUF,
          [pltpu.SemaphoreType.DMA] * _NBUF,
      ],
  )
  def gather_kernel(table_hbm, idx_hbm, out_hbm, idx_stage, dense, rows_v,
                    gsems, osems):
    wid = lax.axis_index("s") * _NC + lax.axis_index("c")
    row_base = wid * _ROWS_PER_W

    per_stage = _STAGE_ROWS * _S

    for q in range(_N_STAGE):
      pltpu.sync_copy(
          idx_hbm.at[pl.ds(row_base + q * _STAGE_ROWS, _STAGE_ROWS)],
          idx_stage)

      def repack(r, carry, q=q):
        dense_row = q * per_stage + r * _S
        for o in (0, _L, 2 * _L, _S - _L):
          vals = idx_stage[r, pl.ds(o, _L)]
          dense[pl.ds(dense_row + o, _L)] = vals
        return carry

      lax.fori_loop(0, _STAGE_ROWS, repack, 0, unroll=False)

    def gather_start(c, b, gsem):
      pltpu.async_copy(
          table_hbm.at[dense.at[pl.ds(c * _CHUNK, _CHUNK)]], rows_v.at[b],
          gsem)

    def gather_wait(c, b, gsem):
      pltpu.make_async_copy(
          table_hbm.at[dense.at[pl.ds(c * _CHUNK, _CHUNK)]], rows_v.at[b],
          gsem).wait()

    def out_start(c, b, osem):
      for m in range(_CHUNK_ROWS):
        pltpu.async_copy(
            rows_v.at[b].at[pl.ds(m * _S, _S), pl.ds(0, _D)],
            out_hbm.at[row_base + c * _CHUNK_ROWS + m], osem)

    def out_wait(c, b, osem):
      for m in range(_CHUNK_ROWS):
        pltpu.make_async_copy(
            rows_v.at[b].at[pl.ds(m * _S, _S), pl.ds(0, _D)],
            out_hbm.at[row_base + c * _CHUNK_ROWS + m], osem).wait()

    for b in range(_NBUF):
      gather_start(b, b, gsems[b])

    def step(i, carry):
      c0 = i * _NBUF
      for b in range(_NBUF):
        gather_wait(c0 + b, b, gsems[b])
        out_start(c0 + b, b, osems[b])
      for b in range(_NBUF):
        out_wait(c0 + b, b, osems[b])

        @pl.when(c0 + b + _NBUF < _N_CHUNKS)
        def _(c=c0 + b, b=b):
          gather_start(c + _NBUF, b, gsems[b])
      return carry

    lax.fori_loop(0, _N_CHUNKS // _NBUF, step, 0, unroll=False)

  return gather_kernel


_gather = _make_gather()


def kernel(idx, table):
  idx_p = jnp.pad(idx, ((0, 0), (0, _SP - _S)))
  table_p = jnp.pad(table, ((0, 0), (0, _TP - _D)))
  return _gather(table_p, idx_p)


# speedup vs baseline: 1.0587x; 1.0587x over previous
import functools

import jax
import jax.numpy as jnp
from jax import lax
from jax.experimental import pallas as pl
from jax.experimental.pallas import tpu as pltpu
from jax.experimental.pallas import tpu_sc as plsc

_B = 16384
_S = 50
_SP = 128
_D = 64
_NC = 2
_NS = 16
_NW = _NC * _NS
_ROWS_PER_W = _B // _NW
_IDX_PER_W = _ROWS_PER_W * _S
_STAGE_ROWS = 16
_N_STAGE = _ROWS_PER_W // _STAGE_ROWS
_CHUNK_ROWS = 4
_CHUNK = _CHUNK_ROWS * _S
_N_CHUNKS = _ROWS_PER_W // _CHUNK_ROWS
_NBUF = 8
_L = 16


def _make_gather():
  mesh = plsc.VectorSubcoreMesh(core_axis_name="c", subcore_axis_name="s")

  @functools.partial(
      pl.kernel,
      out_type=jax.ShapeDtypeStruct((_B, _S, _D), jnp.float32),
      mesh=mesh,
      compiler_params=pltpu.CompilerParams(use_tc_tiling_on_sc=False),
      scratch_types=[
          pltpu.VMEM((_STAGE_ROWS, _SP), jnp.int32),
          pltpu.VMEM((_IDX_PER_W,), jnp.int32),
          pltpu.VMEM((_NBUF, _CHUNK, _D), jnp.float32),
          [pltpu.SemaphoreType.DMA] * _NBUF,
          [pltpu.SemaphoreType.DMA] * _NBUF,
      ],
  )
  def gather_kernel(table_hbm, idx_hbm, out_hbm, idx_stage, dense, rows_v,
                    gsems, osems):
    wid = lax.axis_index("s") * _NC + lax.axis_index("c")
    row_base = wid * _ROWS_PER_W

    per_stage = _STAGE_ROWS * _S

    for q in range(_N_STAGE):
      pltpu.sync_copy(
          idx_hbm.at[pl.ds(row_base + q * _STAGE_ROWS, _STAGE_ROWS)],
          idx_stage)

      def repack(r, carry, q=q):
        dense_row = q * per_stage + r * _S
        for o in (0, _L, 2 * _L, _S - _L):
          vals = idx_stage[r, pl.ds(o, _L)]
          dense[pl.ds(dense_row + o, _L)] = vals
        return carry

      lax.fori_loop(0, _STAGE_ROWS, repack, 0, unroll=False)

    def gather_start(c, b, gsem):
      pltpu.async_copy(
          table_hbm.at[dense.at[pl.ds(c * _CHUNK, _CHUNK)]], rows_v.at[b],
          gsem)

    def gather_wait(c, b, gsem):
      pltpu.make_async_copy(
          table_hbm.at[dense.at[pl.ds(c * _CHUNK, _CHUNK)]], rows_v.at[b],
          gsem).wait()

    def out_start(c, b, osem):
      for m in range(_CHUNK_ROWS):
        pltpu.async_copy(
            rows_v.at[b].at[pl.ds(m * _S, _S)],
            out_hbm.at[row_base + c * _CHUNK_ROWS + m], osem)

    def out_wait(c, b, osem):
      for m in range(_CHUNK_ROWS):
        pltpu.make_async_copy(
            rows_v.at[b].at[pl.ds(m * _S, _S)],
            out_hbm.at[row_base + c * _CHUNK_ROWS + m], osem).wait()

    for b in range(_NBUF):
      gather_start(b, b, gsems[b])

    def step(i, carry):
      c0 = i * _NBUF
      for b in range(_NBUF):
        gather_wait(c0 + b, b, gsems[b])
        out_start(c0 + b, b, osems[b])
      for b in range(_NBUF):
        out_wait(c0 + b, b, osems[b])

        @pl.when(c0 + b + _NBUF < _N_CHUNKS)
        def _(c=c0 + b, b=b):
          gather_start(c + _NBUF, b, gsems[b])
      return carry

    lax.fori_loop(0, _N_CHUNKS // _NBUF, step, 0, unroll=False)

  return gather_kernel


_gather = _make_gather()


def kernel(idx, table):
  idx_p = jnp.pad(idx, ((0, 0), (0, _SP - _S)))
  return _gather(table, idx_p)


# speedup vs baseline: 1.0730x; 1.0136x over previous
import functools

import jax
import jax.numpy as jnp
from jax import lax
from jax.experimental import pallas as pl
from jax.experimental.pallas import tpu as pltpu
from jax.experimental.pallas import tpu_sc as plsc

_B = 16384
_S = 50
_SP = 128
_D = 64
_NC = 2
_NS = 16
_NW = _NC * _NS
_ROWS_PER_W = _B // _NW
_IDX_PER_W = _ROWS_PER_W * _S
_STAGE_ROWS = 128
_N_STAGE = _ROWS_PER_W // _STAGE_ROWS
_CHUNK_ROWS = 4
_CHUNK = _CHUNK_ROWS * _S
_N_CHUNKS = _ROWS_PER_W // _CHUNK_ROWS
_NBUF = 4
_L = 16


def _make_gather():
  mesh = plsc.VectorSubcoreMesh(core_axis_name="c", subcore_axis_name="s")

  @functools.partial(
      pl.kernel,
      out_type=jax.ShapeDtypeStruct((_B, _S, _D), jnp.float32),
      mesh=mesh,
      compiler_params=pltpu.CompilerParams(use_tc_tiling_on_sc=False),
      scratch_types=[
          pltpu.VMEM((_STAGE_ROWS, _SP), jnp.int32),
          pltpu.VMEM((_IDX_PER_W,), jnp.int32),
          pltpu.VMEM((_NBUF, _CHUNK, _D), jnp.float32),
          [pltpu.SemaphoreType.DMA] * _NBUF,
          [pltpu.SemaphoreType.DMA] * _NBUF,
      ],
  )
  def gather_kernel(table_hbm, idx_hbm, out_hbm, idx_stage, dense, rows_v,
                    gsems, osems):
    wid = lax.axis_index("s") * _NC + lax.axis_index("c")
    row_base = wid * _ROWS_PER_W

    per_stage = _STAGE_ROWS * _S

    for q in range(_N_STAGE):
      pltpu.sync_copy(
          idx_hbm.at[pl.ds(row_base + q * _STAGE_ROWS, _STAGE_ROWS)],
          idx_stage)

      def repack(r, carry, q=q):
        dense_row = q * per_stage + r * _S
        for o in (0, _L, 2 * _L, _S - _L):
          vals = idx_stage[r, pl.ds(o, _L)]
          dense[pl.ds(dense_row + o, _L)] = vals
        return carry

      lax.fori_loop(0, _STAGE_ROWS, repack, 0, unroll=False)

    def gather_start(c, b, gsem):
      pltpu.async_copy(
          table_hbm.at[dense.at[pl.ds(c * _CHUNK, _CHUNK)]], rows_v.at[b],
          gsem)

    def gather_wait(c, b, gsem):
      pltpu.make_async_copy(
          table_hbm.at[dense.at[pl.ds(c * _CHUNK, _CHUNK)]], rows_v.at[b],
          gsem).wait()

    def out_start(c, b, osem):
      for m in range(_CHUNK_ROWS):
        pltpu.async_copy(
            rows_v.at[b].at[pl.ds(m * _S, _S)],
            out_hbm.at[row_base + c * _CHUNK_ROWS + m], osem)

    def out_wait(c, b, osem):
      for m in range(_CHUNK_ROWS):
        pltpu.make_async_copy(
            rows_v.at[b].at[pl.ds(m * _S, _S)],
            out_hbm.at[row_base + c * _CHUNK_ROWS + m], osem).wait()

    for b in range(_NBUF):
      gather_start(b, b, gsems[b])

    def step(i, carry):
      c0 = i * _NBUF
      for b in range(_NBUF):
        gather_wait(c0 + b, b, gsems[b])
        out_start(c0 + b, b, osems[b])
      for b in range(_NBUF):
        out_wait(c0 + b, b, osems[b])

        @pl.when(c0 + b + _NBUF < _N_CHUNKS)
        def _(c=c0 + b, b=b):
          gather_start(c + _NBUF, b, gsems[b])
      return carry

    lax.fori_loop(0, _N_CHUNKS // _NBUF, step, 0, unroll=False)

  return gather_kernel


_gather = _make_gather()


def kernel(idx, table):
  idx_p = jnp.pad(idx, ((0, 0), (0, _SP - _S)))
  return _gather(table, idx_p)
